# baseline (speedup 1.0000x reference)
import functools

import jax
import jax.numpy as jnp
from jax import lax
from jax.experimental import pallas as pl
from jax.experimental.pallas import tpu as pltpu
from jax.experimental.pallas import tpu_sc as plsc

N = 50000
E = 800000
NW = 32
K = 128
E_PAD = 802816
EPW = E_PAD // NW
NCH_G = EPW // K
R_DST = 1568
N_OUT = NW * R_DST + 8


def _ab_body(h_ref, w_ref, ba_ref, a_ref, b_ref, *, cout, relu):
    h = h_ref[...]
    if relu:
        h = jnp.maximum(h, 0.0)
    p = jnp.dot(h, w_ref[...], preferred_element_type=jnp.float32,
                precision=lax.Precision.HIGHEST)
    a_ref[...] = p[:, :cout] + ba_ref[...]
    b_ref[...] = p[:, cout:]


def _ab_call(h, w, ba, *, relu):
    cin, c2 = w.shape
    cout = c2 // 2
    bn = 2000
    return pl.pallas_call(
        functools.partial(_ab_body, cout=cout, relu=relu),
        grid=(N // bn,),
        in_specs=[
            pl.BlockSpec((bn, cin), lambda i: (i, 0)),
            pl.BlockSpec((cin, c2), lambda i: (0, 0)),
            pl.BlockSpec((1, cout), lambda i: (0, 0)),
        ],
        out_specs=[
            pl.BlockSpec((bn, cout), lambda i: (i, 0)),
            pl.BlockSpec((bn, cout), lambda i: (i, 0)),
        ],
        out_shape=[
            jax.ShapeDtypeStruct((N, cout), jnp.float32),
            jax.ShapeDtypeStruct((N, cout), jnp.float32),
        ],
    )(h, w, ba.reshape(1, cout))


def _ab0_body(h_ref, w_ref, bias_ref, t_ref):
    t_ref[...] = (
        jnp.dot(h_ref[...], w_ref[...], preferred_element_type=jnp.float32,
                precision=lax.Precision.HIGHEST)
        + bias_ref[...]
    )


def _ab0_call(h, w, ba):
    cin, c2 = w.shape
    bn = 2000
    bias = jnp.concatenate([ba, jnp.zeros((c2 - ba.shape[0],), jnp.float32)])
    return pl.pallas_call(
        _ab0_body,
        grid=(N // bn,),
        in_specs=[
            pl.BlockSpec((bn, cin), lambda i: (i, 0)),
            pl.BlockSpec((cin, c2), lambda i: (0, 0)),
            pl.BlockSpec((1, c2), lambda i: (0, 0)),
        ],
        out_specs=pl.BlockSpec((bn, c2), lambda i: (i, 0)),
        out_shape=jax.ShapeDtypeStruct((N, c2), jnp.float32),
    )(h, w, bias.reshape(1, c2))


def _m_body(pre_ref, wb_ref, bb_ref, m_ref):
    m_ref[...] = (
        jnp.dot(jnp.maximum(pre_ref[...], 0.0), wb_ref[...],
                preferred_element_type=jnp.float32,
                precision=lax.Precision.HIGHEST)
        + bb_ref[...]
    )


def _m_call(pre, wb, bb):
    c = wb.shape[0]
    be = 2048
    return pl.pallas_call(
        _m_body,
        grid=(E_PAD // be,),
        in_specs=[
            pl.BlockSpec((be, c), lambda i: (i, 0)),
            pl.BlockSpec((c, c), lambda i: (0, 0)),
            pl.BlockSpec((1, c), lambda i: (0, 0)),
        ],
        out_specs=pl.BlockSpec((be, c), lambda i: (i, 0)),
        out_shape=jax.ShapeDtypeStruct((E_PAD, c), jnp.float32),
    )(pre, wb, bb.reshape(1, c))


def _m0_body(pre_ref, wb_ref, bb_ref, m_ref, *, cout):
    be = pre_ref.shape[0]
    m = (
        jnp.dot(jnp.maximum(pre_ref[:, :cout], 0.0), wb_ref[...],
                preferred_element_type=jnp.float32,
                precision=lax.Precision.HIGHEST)
        + bb_ref[...]
    )
    m_ref[...] = jnp.concatenate(
        [m, jnp.zeros((be, 128 - cout), jnp.float32)], axis=1)


def _m0_call(pre, wb, bb):
    c = wb.shape[0]
    be = 2048
    return pl.pallas_call(
        functools.partial(_m0_body, cout=c),
        grid=(E_PAD // be,),
        in_specs=[
            pl.BlockSpec((be, 128), lambda i: (i, 0)),
            pl.BlockSpec((c, c), lambda i: (0, 0)),
            pl.BlockSpec((1, c), lambda i: (0, 0)),
        ],
        out_specs=pl.BlockSpec((be, 128), lambda i: (i, 0)),
        out_shape=jax.ShapeDtypeStruct((E_PAD, 128), jnp.float32),
    )(pre, wb, bb.reshape(1, c))


_MESH = plsc.VectorSubcoreMesh(core_axis_name="c", subcore_axis_name="s")


def _worker_id():
    return lax.axis_index("s") * 2 + lax.axis_index("c")


def _make_gather(c):
    @functools.partial(
        pl.kernel,
        out_type=jax.ShapeDtypeStruct((E_PAD, c), jnp.float32),
        mesh=_MESH,
        compiler_params=pltpu.CompilerParams(needs_layout_passes=False),
        scratch_types=[
            pltpu.VMEM((K,), jnp.int32),
            pltpu.VMEM((K,), jnp.int32),
            pltpu.VMEM((K, c), jnp.float32),
            pltpu.VMEM((K, c), jnp.float32),
            pltpu.VMEM((K, c), jnp.float32),
            pltpu.SemaphoreType.DMA,
            pltpu.SemaphoreType.DMA,
        ],
    )
    def gather_k(a_hbm, b_hbm, dst_hbm, src_hbm, pre_hbm,
                 di, si, ar, br, orows, sem_a, sem_b):
        wid = _worker_id()
        base0 = wid * EPW

        def chunk(k, carry):
            base = base0 + k * K
            pltpu.sync_copy(dst_hbm.at[pl.ds(base, K)], di)
            pltpu.sync_copy(src_hbm.at[pl.ds(base, K)], si)
            cp_a = pltpu.async_copy(a_hbm.at[di], ar, sem_a)
            cp_b = pltpu.async_copy(b_hbm.at[si], br, sem_b)
            cp_a.wait()
            cp_b.wait()

            def row(r, _):
                for j in range(c // 16):
                    s = pl.ds(j * 16, 16)
                    orows[r, s] = ar[r, s] + br[r, s]
                return 0

            lax.fori_loop(0, K, row, 0)
            pltpu.sync_copy(orows, pre_hbm.at[pl.ds(base, K)])
            return carry

        lax.fori_loop(0, NCH_G, chunk, 0)

    return gather_k


def _make_gather0():
    @functools.partial(
        pl.kernel,
        out_type=jax.ShapeDtypeStruct((E_PAD, 128), jnp.float32),
        mesh=_MESH,
        compiler_params=pltpu.CompilerParams(needs_layout_passes=False),
        scratch_types=[
            pltpu.VMEM((K,), jnp.int32),
            pltpu.VMEM((K,), jnp.int32),
            pltpu.VMEM((K, 128), jnp.float32),
            pltpu.VMEM((K, 128), jnp.float32),
            pltpu.VMEM((K, 128), jnp.float32),
            pltpu.SemaphoreType.DMA,
            pltpu.SemaphoreType.DMA,
        ],
    )
    def gather0_k(t_hbm, dst_hbm, src_hbm, pre_hbm,
                  di, si, td, ts, orows, sem_a, sem_b):
        wid = _worker_id()
        base0 = wid * EPW

        def chunk(k, carry):
            base = base0 + k * K
            pltpu.sync_copy(dst_hbm.at[pl.ds(base, K)], di)
            pltpu.sync_copy(src_hbm.at[pl.ds(base, K)], si)
            cp_a = pltpu.async_copy(t_hbm.at[di], td, sem_a)
            cp_b = pltpu.async_copy(t_hbm.at[si], ts, sem_b)
            cp_a.wait()
            cp_b.wait()

            def row(r, _):
                for j in range(4):
                    s = pl.ds(j * 16, 16)
                    orows[r, s] = td[r, s] + ts[r, pl.ds(64 + j * 16, 16)]
                return 0

            lax.fori_loop(0, K, row, 0)
            pltpu.sync_copy(orows, pre_hbm.at[pl.ds(base, K)])
            return carry

        lax.fori_loop(0, NCH_G, chunk, 0)

    return gather0_k


def _make_segmax(c):
    nv = c // 16

    @functools.partial(
        pl.kernel,
        out_type=jax.ShapeDtypeStruct((N_OUT, c), jnp.float32),
        mesh=_MESH,
        compiler_params=pltpu.CompilerParams(needs_layout_passes=False),
        scratch_types=[
            pltpu.VMEM((K * c,), jnp.float32),
            pltpu.VMEM((K,), jnp.int32),
            pltpu.VMEM((K, c), jnp.float32),
            pltpu.VMEM((K,), jnp.int32),
            pltpu.VMEM((48,), jnp.int32),
            pltpu.SemaphoreType.DMA,
        ],
    )
    def segmax_k(m_hbm, dst_hbm, starts_hbm, out_hbm,
                 mrows, dv, rows, idxb, svv, sem):
        wid = _worker_id()
        iota = lax.iota(jnp.int32, 16)
        lane0 = iota == 0
        pltpu.sync_copy(starts_hbm.at[pl.ds(0, 48)], svv)

        def rd(idx):
            r = jnp.int32(0)
            for t in range(3):
                v = svv[pl.ds(t * 16, 16)]
                msk = (iota + t * 16) == jnp.broadcast_to(idx, (16,))
                r = jnp.maximum(r, jnp.max(jnp.where(msk, v, 0), axis=0))
            return r

        s0 = rd(wid)
        s1 = rd(wid + 1)

        def zrow(r, _):
            for j in range(nv):
                rows[r, pl.ds(j * 16, 16)] = jnp.zeros((16,), jnp.float32)
            return 0

        lax.fori_loop(0, K, zrow, 0)
        obase = wid * R_DST
        for t in range(12):
            pltpu.sync_copy(rows, out_hbm.at[pl.ds(obase + t * K, K)])
        pltpu.sync_copy(rows.at[pl.ds(0, 32)],
                        out_hbm.at[pl.ds(obase + 12 * K, 32)])

        trash = jnp.broadcast_to(
            jnp.int32(NW * R_DST) + lax.rem(wid, 8), (16,))
        s0a = (s0 // 8) * 8
        nch = (s1 - s0a + K - 1) // K

        def chunk(k, carry):
            dprev, acc = carry
            base = s0a + k * K
            pltpu.sync_copy(m_hbm.at[pl.ds(base * c, K * c)], mrows)
            pltpu.sync_copy(dst_hbm.at[pl.ds(base, K)], dv)
            for t in range(K // 16):
                idxb[pl.ds(t * 16, 16)] = trash
            ilo = jnp.maximum(s0 - base, 0)
            ihi = jnp.minimum(s1 - base, K)
            ilo_v = jnp.broadcast_to(ilo, (16,))
            cnt0 = jnp.full((16,), -1, jnp.int32)

            def edge(i, ec):
                cnt, dprev, acc = ec
                iv = jnp.broadcast_to(i, (16,))
                dvec = plsc.load_gather(dv, [iv])
                new = dvec != dprev
                cnt = cnt + jnp.where(new | (iv == ilo_v), 1, 0)
                plsc.store_scatter(idxb, [cnt], dvec, mask=lane0)
                pen = jnp.where(new, jnp.float32(-3.4e38), jnp.float32(0.0))
                nacc = []
                for j in range(nv):
                    mrow = mrows[pl.ds(i * c + j * 16, 16)]
                    aj = jnp.maximum(acc[j] + pen, mrow)
                    plsc.store_scatter(rows, [cnt, iota + j * 16], aj)
                    nacc.append(aj)
                return cnt, dvec, tuple(nacc)

            _, dprev, acc = lax.fori_loop(ilo, ihi, edge,
                                          (cnt0, dprev, acc))
            pltpu.async_copy(rows, out_hbm.at[idxb], sem).wait()
            return dprev, acc

        dprev0 = jnp.full((16,), -1, jnp.int32)
        acc0 = tuple(jnp.zeros((16,), jnp.float32) for _ in range(nv))
        lax.fori_loop(0, nch, chunk, (dprev0, acc0))

    return segmax_k


_GATHER0 = _make_gather0()
_GATHER128 = _make_gather(128)
_SEGMAX128 = _make_segmax(128)


def kernel(x, Wa0, ba0, Wb0, bb0, Wa1, ba1, Wb1, bb1,
           Wa2, ba2, Wb2, bb2, Wa3, ba3, Wb3, bb3, edge_index):
    src = edge_index[0]
    dst = edge_index[1]
    perm = jnp.argsort(dst)
    dsts = jnp.concatenate(
        [dst[perm], jnp.full((E_PAD - E,), N, jnp.int32)])
    srcs = jnp.concatenate(
        [src[perm], jnp.zeros((E_PAD - E,), jnp.int32)])
    dstg = jnp.minimum(dsts, N - 1)
    bounds = jnp.minimum(jnp.arange(33, dtype=jnp.int32) * R_DST, N)
    starts = jnp.searchsorted(dsts, bounds, side='left').astype(jnp.int32)
    starts = jnp.concatenate([starts, jnp.zeros((15,), jnp.int32)])

    params = [(Wa0, ba0, Wb0, bb0), (Wa1, ba1, Wb1, bb1),
              (Wa2, ba2, Wb2, bb2), (Wa3, ba3, Wb3, bb3)]
    h = x
    for i, (Wa, ba, Wb, bb) in enumerate(params):
        cin = Wa.shape[0] // 2
        wcomb = jnp.concatenate([Wa[:cin] - Wa[cin:], Wa[cin:]], axis=1)
        if i == 0:
            t = _ab0_call(h, wcomb, ba)
            pre = _GATHER0(t, dstg, srcs)
            m = _m0_call(pre, Wb, bb)
        else:
            a, b = _ab_call(h[:, :cin], wcomb, ba, relu=True)
            pre = _GATHER128(a, b, dstg, srcs)
            m = _m_call(pre, Wb, bb)
        out = _SEGMAX128(m.reshape(-1), dsts, starts)
        h = out[:N]
    return h


# speedup vs baseline: 1.9527x; 1.9527x over previous
import functools

import jax
import jax.numpy as jnp
from jax import lax
from jax.experimental import pallas as pl
from jax.experimental.pallas import tpu as pltpu
from jax.experimental.pallas import tpu_sc as plsc

N = 50000
E = 800000
NW = 32
K = 128
E_PAD = 802816
EPW = E_PAD // NW
NCH_G = EPW // K
R_DST = 1568
N_OUT = NW * R_DST + 8


def _ab_body(h_ref, w_ref, ba_ref, a_ref, b_ref, *, cout, relu):
    h = h_ref[...]
    if relu:
        h = jnp.maximum(h, 0.0)
    p = jnp.dot(h, w_ref[...], preferred_element_type=jnp.float32,
                precision=lax.Precision.HIGHEST)
    a_ref[...] = p[:, :cout] + ba_ref[...]
    b_ref[...] = p[:, cout:]


def _ab_call(h, w, ba, *, relu):
    cin, c2 = w.shape
    cout = c2 // 2
    bn = 2000
    return pl.pallas_call(
        functools.partial(_ab_body, cout=cout, relu=relu),
        grid=(N // bn,),
        in_specs=[
            pl.BlockSpec((bn, cin), lambda i: (i, 0)),
            pl.BlockSpec((cin, c2), lambda i: (0, 0)),
            pl.BlockSpec((1, cout), lambda i: (0, 0)),
        ],
        out_specs=[
            pl.BlockSpec((bn, cout), lambda i: (i, 0)),
            pl.BlockSpec((bn, cout), lambda i: (i, 0)),
        ],
        out_shape=[
            jax.ShapeDtypeStruct((N, cout), jnp.float32),
            jax.ShapeDtypeStruct((N, cout), jnp.float32),
        ],
    )(h, w, ba.reshape(1, cout))


def _ab0_body(h_ref, w_ref, bias_ref, t_ref):
    t_ref[...] = (
        jnp.dot(h_ref[...], w_ref[...], preferred_element_type=jnp.float32,
                precision=lax.Precision.HIGHEST)
        + bias_ref[...]
    )


def _ab0_call(h, w, ba):
    cin, c2 = w.shape
    bn = 2000
    bias = jnp.concatenate([ba, jnp.zeros((c2 - ba.shape[0],), jnp.float32)])
    return pl.pallas_call(
        _ab0_body,
        grid=(N // bn,),
        in_specs=[
            pl.BlockSpec((bn, cin), lambda i: (i, 0)),
            pl.BlockSpec((cin, c2), lambda i: (0, 0)),
            pl.BlockSpec((1, c2), lambda i: (0, 0)),
        ],
        out_specs=pl.BlockSpec((bn, c2), lambda i: (i, 0)),
        out_shape=jax.ShapeDtypeStruct((N, c2), jnp.float32),
    )(h, w, bias.reshape(1, c2))


def _m_body(pre_ref, wb_ref, bb_ref, m_ref):
    m_ref[...] = (
        jnp.dot(jnp.maximum(pre_ref[...], 0.0), wb_ref[...],
                preferred_element_type=jnp.float32,
                precision=lax.Precision.HIGHEST)
        + bb_ref[...]
    )


def _m_call(pre, wb, bb):
    c = wb.shape[0]
    be = 2048
    return pl.pallas_call(
        _m_body,
        grid=(E_PAD // be,),
        in_specs=[
            pl.BlockSpec((be, c), lambda i: (i, 0)),
            pl.BlockSpec((c, c), lambda i: (0, 0)),
            pl.BlockSpec((1, c), lambda i: (0, 0)),
        ],
        out_specs=pl.BlockSpec((be, c), lambda i: (i, 0)),
        out_shape=jax.ShapeDtypeStruct((E_PAD, c), jnp.float32),
    )(pre, wb, bb.reshape(1, c))


def _m0_body(pre_ref, wb_ref, bb_ref, m_ref, *, cout):
    be = pre_ref.shape[0]
    m = (
        jnp.dot(jnp.maximum(pre_ref[:, :cout], 0.0), wb_ref[...],
                preferred_element_type=jnp.float32,
                precision=lax.Precision.HIGHEST)
        + bb_ref[...]
    )
    m_ref[...] = jnp.concatenate(
        [m, jnp.zeros((be, 128 - cout), jnp.float32)], axis=1)


def _m0_call(pre, wb, bb):
    c = wb.shape[0]
    be = 2048
    return pl.pallas_call(
        functools.partial(_m0_body, cout=c),
        grid=(E_PAD // be,),
        in_specs=[
            pl.BlockSpec((be, 128), lambda i: (i, 0)),
            pl.BlockSpec((c, c), lambda i: (0, 0)),
            pl.BlockSpec((1, c), lambda i: (0, 0)),
        ],
        out_specs=pl.BlockSpec((be, 128), lambda i: (i, 0)),
        out_shape=jax.ShapeDtypeStruct((E_PAD, 128), jnp.float32),
    )(pre, wb, bb.reshape(1, c))


_MESH = plsc.VectorSubcoreMesh(core_axis_name="c", subcore_axis_name="s")


def _worker_id():
    return lax.axis_index("s") * 2 + lax.axis_index("c")


def _make_gather(c, combined):
    nj = 4 if combined else c // 16

    @functools.partial(
        pl.kernel,
        out_type=jax.ShapeDtypeStruct((E_PAD, c), jnp.float32),
        mesh=_MESH,
        compiler_params=pltpu.CompilerParams(needs_layout_passes=False),
        scratch_types=(
            [pltpu.VMEM((K,), jnp.int32) for _ in range(4)]
            + [pltpu.VMEM((K, c), jnp.float32) for _ in range(5)]
            + [pltpu.SemaphoreType.DMA for _ in range(4)]
        ),
    )
    def gather_k(*args):
        na = 4 if combined else 5
        if combined:
            a_hbm = b_hbm = args[0]
            dst_hbm, src_hbm, pre_hbm = args[1:4]
        else:
            a_hbm, b_hbm, dst_hbm, src_hbm, pre_hbm = args[:5]
        rest = args[na:]
        di, si = rest[0:2], rest[2:4]
        ar, br = rest[4:6], rest[6:8]
        orows = rest[8]
        sa, sb = rest[9:11], rest[11:13]
        wid = _worker_id()
        base0 = wid * EPW

        def fire(ck, b):
            base = base0 + ck * K
            pltpu.sync_copy(dst_hbm.at[pl.ds(base, K)], di[b])
            pltpu.sync_copy(src_hbm.at[pl.ds(base, K)], si[b])
            pltpu.async_copy(a_hbm.at[di[b]], ar[b], sa[b])
            pltpu.async_copy(b_hbm.at[si[b]], br[b], sb[b])

        def wait(b):
            pltpu.make_async_copy(a_hbm.at[di[b]], ar[b], sa[b]).wait()
            pltpu.make_async_copy(b_hbm.at[si[b]], br[b], sb[b]).wait()

        fire(jnp.int32(0), 0)

        def big(k2, carry):
            for b in (0, 1):
                cur = 2 * k2 + b
                fire(jnp.minimum(cur + 1, NCH_G - 1), 1 - b)
                wait(b)

                def row(r, _):
                    for j in range(nj):
                        sl = pl.ds(j * 16, 16)
                        sr = pl.ds(64 + j * 16, 16) if combined else sl
                        orows[r, sl] = ar[b][r, sl] + br[b][r, sr]
                    return 0

                lax.fori_loop(0, K, row, 0)
                pltpu.sync_copy(orows, pre_hbm.at[pl.ds(base0 + cur * K, K)])
            return carry

        lax.fori_loop(0, NCH_G // 2, big, 0)
        wait(0)

    return gather_k


def _make_segmax(c):
    nv = c // 16
    KS = 256
    NG = KS // 16

    @functools.partial(
        pl.kernel,
        out_type=jax.ShapeDtypeStruct((N_OUT, c), jnp.float32),
        mesh=_MESH,
        compiler_params=pltpu.CompilerParams(needs_layout_passes=False),
        scratch_types=[
            pltpu.VMEM((KS * c,), jnp.float32),
            pltpu.VMEM((KS,), jnp.int32),
            pltpu.VMEM((KS, c), jnp.float32),
            pltpu.VMEM((NG, 16), jnp.int32),
            pltpu.VMEM((48,), jnp.int32),
            pltpu.SemaphoreType.DMA,
        ],
    )
    def segmax_k(m_hbm, dst_hbm, starts_hbm, out_hbm,
                 mrows, dv, rows, idxb, svv, sem):
        wid = _worker_id()
        iota = lax.iota(jnp.int32, 16)
        lane0 = iota == 0
        pltpu.sync_copy(starts_hbm.at[pl.ds(0, 48)], svv)

        def rd(idx):
            r = jnp.int32(0)
            for t in range(3):
                v = svv[pl.ds(t * 16, 16)]
                msk = (iota + t * 16) == jnp.broadcast_to(idx, (16,))
                r = jnp.maximum(r, jnp.max(jnp.where(msk, v, 0), axis=0))
            return r

        s0 = rd(wid)
        s1 = rd(wid + 1)

        def zrow(r, _):
            for j in range(nv):
                rows[r, pl.ds(j * 16, 16)] = jnp.zeros((16,), jnp.float32)
            return 0

        lax.fori_loop(0, KS, zrow, 0)
        obase = wid * R_DST
        for t in range(R_DST // KS):
            pltpu.sync_copy(rows, out_hbm.at[pl.ds(obase + t * KS, KS)])
        pltpu.sync_copy(rows.at[pl.ds(0, R_DST % KS)],
                        out_hbm.at[pl.ds(obase + (R_DST // KS) * KS,
                                         R_DST % KS)])

        trash = jnp.broadcast_to(
            jnp.int32(NW * R_DST) + lax.rem(wid, 8), (16,))
        s0a = (s0 // 8) * 8
        nch = (s1 - s0a + KS - 1) // KS

        def chunk(k, carry):
            dprev, acc = carry
            base = s0a + k * KS
            pltpu.sync_copy(m_hbm.at[pl.ds(base * c, KS * c)], mrows)
            pltpu.sync_copy(dst_hbm.at[pl.ds(base, KS)], dv)
            for g in range(NG):
                idxb[g, :] = trash
            ilo = jnp.maximum(s0 - base, 0)
            ihi = jnp.minimum(s1 - base, KS)
            ilo_v = jnp.broadcast_to(ilo, (16,))
            cnt0 = jnp.full((16,), -1, jnp.int32)

            def edge(i, ec):
                cnt, dprev, acc = ec
                iv = jnp.broadcast_to(i, (16,))
                dvec = plsc.load_gather(dv, [iv])
                new = dvec != dprev
                cnt = cnt + jnp.where(new | (iv == ilo_v), 1, 0)
                plsc.store_scatter(idxb, [cnt // 16, lax.rem(cnt, 16)],
                                   dvec, mask=lane0)
                pen = jnp.where(new, jnp.float32(-3.4e38), jnp.float32(0.0))
                nacc = []
                for j in range(nv):
                    mrow = mrows[pl.ds(i * c + j * 16, 16)]
                    aj = jnp.maximum(acc[j] + pen, mrow)
                    plsc.store_scatter(rows, [cnt, iota + j * 16], aj)
                    nacc.append(aj)
                return cnt, dvec, tuple(nacc)

            cnt, dprev, acc = lax.fori_loop(ilo, ihi, edge,
                                            (cnt0, dprev, acc))
            cs = jnp.max(cnt, axis=0)
            for g in range(NG):
                @pl.when(cs >= g * 16)
                def _flush():
                    pltpu.async_copy(rows.at[pl.ds(g * 16, 16)],
                                     out_hbm.at[idxb.at[g]], sem).wait()
            return dprev, acc

        dprev0 = jnp.full((16,), -1, jnp.int32)
        acc0 = tuple(jnp.zeros((16,), jnp.float32) for _ in range(nv))
        lax.fori_loop(0, nch, chunk, (dprev0, acc0))

    return segmax_k


_GATHER0 = _make_gather(128, True)
_GATHER128 = _make_gather(128, False)
_SEGMAX128 = _make_segmax(128)


def kernel(x, Wa0, ba0, Wb0, bb0, Wa1, ba1, Wb1, bb1,
           Wa2, ba2, Wb2, bb2, Wa3, ba3, Wb3, bb3, edge_index):
    src = edge_index[0]
    dst = edge_index[1]
    perm = jnp.argsort(dst)
    dsts = jnp.concatenate(
        [dst[perm], jnp.full((E_PAD - E,), N, jnp.int32)])
    srcs = jnp.concatenate(
        [src[perm], jnp.zeros((E_PAD - E,), jnp.int32)])
    dstg = jnp.minimum(dsts, N - 1)
    bounds = jnp.minimum(jnp.arange(33, dtype=jnp.int32) * R_DST, N)
    starts = jnp.searchsorted(dsts, bounds, side='left').astype(jnp.int32)
    starts = jnp.concatenate([starts, jnp.zeros((15,), jnp.int32)])

    params = [(Wa0, ba0, Wb0, bb0), (Wa1, ba1, Wb1, bb1),
              (Wa2, ba2, Wb2, bb2), (Wa3, ba3, Wb3, bb3)]
    h = x
    for i, (Wa, ba, Wb, bb) in enumerate(params):
        cin = Wa.shape[0] // 2
        wcomb = jnp.concatenate([Wa[:cin] - Wa[cin:], Wa[cin:]], axis=1)
        if i == 0:
            t = _ab0_call(h, wcomb, ba)
            pre = _GATHER0(t, dstg, srcs)
            m = _m0_call(pre, Wb, bb)
        else:
            a, b = _ab_call(h[:, :cin], wcomb, ba, relu=True)
            pre = _GATHER128(a, b, dstg, srcs)
            m = _m_call(pre, Wb, bb)
        out = _SEGMAX128(m.reshape(-1), dsts, starts)
        h = out[:N]
    return h
